# baseline (speedup 1.0000x reference)
import functools

import jax
import jax.numpy as jnp
from jax import lax
from jax.experimental import pallas as pl
from jax.experimental.pallas import tpu as pltpu
from jax.experimental.pallas import tpu_sc as plsc

VOCAB = 100000
GOVS = 50
EMBED = 16
BATCH = 1024
CTX = 40

_IDX_CHUNK = 128
_VT = 2048
_NCHK = 781


def _pack_sc(wt, tail):
    info = plsc.get_sparse_core_info()
    nc, ns = info.num_cores, info.num_subcores
    nw = nc * ns
    ntmax = -(-_NCHK // nw)
    nbig = _NCHK - nw * (ntmax - 1)
    mesh = plsc.VectorSubcoreMesh(core_axis_name="c", subcore_axis_name="s")

    @functools.partial(
        pl.kernel,
        out_type=jax.ShapeDtypeStruct((VOCAB * EMBED,), jnp.float32),
        mesh=mesh,
        scratch_types=[
            pltpu.VMEM((4, 16, 128), jnp.float32),
            pltpu.VMEM((ntmax * 16 * 128,), jnp.float32),
            pltpu.VMEM((32 * EMBED,), jnp.float32),
            pltpu.SemaphoreType.DMA,
            pltpu.SemaphoreType.DMA,
            pltpu.SemaphoreType.DMA,
            pltpu.SemaphoreType.DMA,
        ],
        compiler_params=pltpu.CompilerParams(needs_layout_passes=False),
    )
    def pack(wt_hbm, tail_hbm, out_hbm, in_v, out_v, tail_v, s0, s1, s2, s3):
        wid = lax.axis_index("s") * nc + lax.axis_index("c")
        big = wid < nbig
        a = jnp.where(big, ntmax * wid,
                      ntmax * nbig + (ntmax - 1) * (wid - nbig))
        nt = jnp.where(big, ntmax, ntmax - 1)
        sems = (s0, s1, s2, s3)
        lanes0 = lax.broadcasted_iota(jnp.int32, (16,), 0)

        def dma(t, start):
            k = a + t
            slot = t % 4
            cp = pltpu.make_async_copy(
                wt_hbm.at[:, pl.ds(128 * k, 128)], in_v.at[slot], sems[slot])

            @pl.when(t < nt)
            def _():
                cp.start() if start else cp.wait()

        for t in range(3):
            dma(t, True)
        for t in range(ntmax):
            if t + 3 < ntmax:
                dma(t + 3, True)
            dma(t, False)

            @pl.when(t < nt)
            def _():
                @plsc.parallel_loop(0, 16)
                def _jloop(j):
                    for p in range(8):
                        v = plsc.load_gather(
                            in_v, [jnp.full((16,), t % 4, jnp.int32),
                                   lanes0,
                                   jnp.zeros((16,), jnp.int32) + (j * 8 + p)])
                        out_v[pl.ds(t * 2048 + j * 128 + p * 16, 16)] = v

        @pl.when(big)
        def _():
            pltpu.sync_copy(out_v, out_hbm.at[pl.ds(2048 * a, ntmax * 2048)])

        @pl.when(jnp.logical_not(big))
        def _():
            pltpu.sync_copy(out_v.at[pl.ds(0, (ntmax - 1) * 2048)],
                            out_hbm.at[pl.ds(2048 * a, (ntmax - 1) * 2048)])

        @pl.when(wid == nw - 1)
        def _():
            pltpu.sync_copy(tail_hbm, tail_v)
            pltpu.sync_copy(tail_v, out_hbm.at[pl.ds(_NCHK * 2048, 512)])

    return pack(wt, tail)


def _combine_sc(context, gov, wl, gov_emb):
    info = plsc.get_sparse_core_info()
    nc, ns = info.num_cores, info.num_subcores
    nw = nc * ns
    bpw = BATCH // nw
    ipw = bpw * CTX
    nch = ipw // _IDX_CHUNK
    ctx_flat = context.reshape(BATCH * CTX)

    mesh = plsc.VectorSubcoreMesh(core_axis_name="c", subcore_axis_name="s")

    @functools.partial(
        pl.kernel,
        out_type=jax.ShapeDtypeStruct((BATCH, EMBED), jnp.float32),
        mesh=mesh,
        scratch_types=[
            pltpu.VMEM((ipw,), jnp.int32),
            pltpu.VMEM((ipw, EMBED), jnp.float32),
            pltpu.VMEM((bpw,), jnp.int32),
            pltpu.VMEM((bpw, EMBED), jnp.float32),
            pltpu.VMEM((bpw, EMBED), jnp.float32),
            pltpu.SemaphoreType.DMA,
        ],
        compiler_params=pltpu.CompilerParams(use_tc_tiling_on_sc=False),
    )
    def combine(ctx_hbm, gov_hbm, wl_hbm, gemb_hbm, out_hbm,
                idx_v, rows_v, gidx_v, grows_v, out_v, sem):
        wid = lax.axis_index("s") * nc + lax.axis_index("c")
        pltpu.sync_copy(ctx_hbm.at[pl.ds(wid * ipw, ipw)], idx_v)
        pltpu.sync_copy(gov_hbm.at[pl.ds(wid * bpw, bpw)], gidx_v)
        copies = [
            pltpu.async_copy(wl_hbm.at[idx_v.at[pl.ds(k * _IDX_CHUNK,
                                                      _IDX_CHUNK)]],
                             rows_v.at[pl.ds(k * _IDX_CHUNK, _IDX_CHUNK)],
                             sem)
            for k in range(nch)
        ]
        copies.append(pltpu.async_copy(gemb_hbm.at[gidx_v], grows_v, sem))
        for c in copies:
            c.wait()

        def row_body(r, _):
            def acc_body(j, acc):
                return acc + rows_v[r * CTX + j, :]
            s = lax.fori_loop(0, CTX, acc_body,
                              jnp.zeros((EMBED,), jnp.float32))
            out_v[r, :] = s * (1.0 / CTX) + grows_v[r, :]
            return 0

        lax.fori_loop(0, bpw, row_body, 0)
        pltpu.sync_copy(out_v, out_hbm.at[pl.ds(wid * bpw, bpw)])

    return combine(ctx_flat, gov, wl, gov_emb)


def _project_tc_t(comb_aug, w_aug_t):
    nvt = pl.cdiv(VOCAB, _VT)
    ka = comb_aug.shape[1]

    def mm(w_ref, comb_ref, out_ref):
        out_ref[...] = lax.dot_general(
            w_ref[...], comb_ref[...],
            dimension_numbers=(((0,), (1,)), ((), ())),
            preferred_element_type=jnp.float32,
        )

    return pl.pallas_call(
        mm,
        grid=(nvt,),
        in_specs=[
            pl.BlockSpec((ka, _VT), lambda i: (0, i)),
            pl.BlockSpec((BATCH, ka), lambda i: (0, 0)),
        ],
        out_specs=pl.BlockSpec((_VT, BATCH), lambda i: (i, 0)),
        out_shape=jax.ShapeDtypeStruct((VOCAB, BATCH), jnp.float32),
    )(w_aug_t, comb_aug)


def kernel(context, gov, word_emb, gov_emb, W, b):
    tail = word_emb[_NCHK * 128:].reshape(32 * EMBED)
    wl = _pack_sc(word_emb.T, tail).reshape(VOCAB, EMBED)
    combined = _combine_sc(context, gov, wl, gov_emb)
    comb_aug = jnp.concatenate(
        [combined, jnp.ones((BATCH, 1), jnp.float32)], axis=1)
    w_aug_t = jnp.concatenate([W.T, b[None, :]], axis=0)
    return _project_tc_t(comb_aug, w_aug_t).T
